# Initial kernel scaffold; baseline (speedup 1.0000x reference)
import jax
import jax.numpy as jnp
from jax.experimental import pallas as pl


def kernel(feat, edge_index, W, b):
    raise NotImplementedError("write your pallas kernel here")


# speedup vs baseline: 5.2249x; 5.2249x over previous
import functools

import jax
import jax.numpy as jnp
from jax import lax
from jax.experimental import pallas as pl
from jax.experimental.pallas import tpu as pltpu
from jax.experimental.pallas import tpu_sc as plsc

_N = 10000
_D = 128
_E = 320000
_NPAD = 10240
_NCORES = 2
_NSUB = 16
_NTILES = _NCORES * _NSUB
_B = 128
_NB = 79
_EPT = _NB * _B
_RPT = _NPAD // _NSUB


@functools.partial(
    pl.kernel,
    mesh=plsc.VectorSubcoreMesh(core_axis_name="c", subcore_axis_name="s"),
    out_type=jax.ShapeDtypeStruct((_NCORES * _NPAD, _D), jnp.float32),
    scratch_types=[
        pltpu.VMEM((_NB, _B), jnp.int32),
        pltpu.VMEM((_NB, _B), jnp.int32),
        pltpu.VMEM((_B, _D), jnp.float32),
        pltpu.SemaphoreType.DMA,
        pltpu.VMEM_SHARED((_NPAD, _D), jnp.float32),
    ],
)
def _edge_pass(u_hbm, col_hbm, row_hbm, agg_hbm, col_v, row_v, buf, sem, acc):
    c = lax.axis_index("c")
    s = lax.axis_index("s")
    w = c * _NSUB + s

    zeros16 = jnp.zeros((16,), jnp.float32)

    def _zrow(i, carry):
        for k in range(_D // 16):
            buf[i, pl.ds(k * 16, 16)] = zeros16
        return carry

    lax.fori_loop(0, _B, _zrow, 0)
    base = s * _RPT
    for k in range(_RPT // _B):
        pltpu.sync_copy(buf, acc.at[pl.ds(base + k * _B, _B)])

    pltpu.sync_copy(col_hbm.at[w], col_v)
    pltpu.sync_copy(row_hbm.at[w], row_v)

    plsc.subcore_barrier()

    def _body(j, carry):
        pltpu.async_copy(u_hbm.at[col_v.at[j]], buf, sem).wait()
        pltpu.sync_copy(buf, acc.at[row_v.at[j]], add=True)
        return carry

    lax.fori_loop(0, _NB, _body, 0)

    plsc.subcore_barrier()

    pltpu.sync_copy(acc.at[pl.ds(base, _RPT)],
                    agg_hbm.at[pl.ds(c * _NPAD + base, _RPT)])


_R = 512
_GRID = _NPAD // _R


def _prep_body(feat_b, agg0_b, agg1_b, u0_b, a_b, c_b, sqd_b):
    deg = agg0_b[...] + agg1_b[...] + 1.0
    dinv = lax.rsqrt(deg)
    u0_b[...] = feat_b[...] * dinv
    inv = 1.0 / deg
    a_b[...] = 0.5 + 0.5 * inv
    c_b[...] = 0.5 * inv
    sqd_b[...] = deg * dinv


def _combine_body(u_b, agg0_b, agg1_b, a_b, c_b, out_b):
    out_b[...] = a_b[...] * u_b[...] + c_b[...] * (agg0_b[...] + agg1_b[...])


def _final_body(u_b, sqd_b, wt_b, bias_b, out_b):
    h = u_b[...] * sqd_b[...]
    out_b[...] = jnp.dot(h, wt_b[...],
                         preferred_element_type=jnp.float32) + bias_b[...]


def _row_spec():
    return pl.BlockSpec((_R, _D), lambda i: (i, 0))


def _agg_specs():
    return [pl.BlockSpec((_R, _D), lambda i: (i, 0)),
            pl.BlockSpec((_R, _D), lambda i: (i + _GRID, 0))]


_prep_call = pl.pallas_call(
    _prep_body,
    grid=(_GRID,),
    in_specs=[_row_spec()] + _agg_specs(),
    out_specs=[_row_spec()] * 4,
    out_shape=[jax.ShapeDtypeStruct((_NPAD, _D), jnp.float32)] * 4,
)

_combine_call = pl.pallas_call(
    _combine_body,
    grid=(_GRID,),
    in_specs=[_row_spec()] + _agg_specs() + [_row_spec(), _row_spec()],
    out_specs=_row_spec(),
    out_shape=jax.ShapeDtypeStruct((_NPAD, _D), jnp.float32),
)

_final_call = pl.pallas_call(
    _final_body,
    grid=(_GRID,),
    in_specs=[_row_spec(), _row_spec(),
              pl.BlockSpec((_D, _D), lambda i: (0, 0)),
              pl.BlockSpec((1, _D), lambda i: (0, 0))],
    out_specs=_row_spec(),
    out_shape=jax.ShapeDtypeStruct((_NPAD, _D), jnp.float32),
)


def kernel(feat, edge_index, W, b):
    row = edge_index[0]
    col = edge_index[1]
    pad = _NTILES * _EPT - _E
    sink = jnp.full((pad,), _NPAD - 1, jnp.int32)
    colp = jnp.concatenate([col, sink]).reshape(_NTILES, _NB, _B)
    rowp = jnp.concatenate([row, sink]).reshape(_NTILES, _NB, _B)

    featp = jnp.pad(feat, ((0, _NPAD - _N), (0, 0)))
    ones = jnp.pad(jnp.ones((_N, _D), jnp.float32), ((0, _NPAD - _N), (0, 0)))

    agg_deg = _edge_pass(ones, colp, rowp)
    u, a, cf, sqd = _prep_call(featp, agg_deg, agg_deg)

    for _ in range(8):
        agg = _edge_pass(u, colp, rowp)
        u = _combine_call(u, agg, agg, a, cf)

    out = _final_call(u, sqd, W.T, b.reshape(1, _D))
    return out[:_N]
